# baseline (speedup 1.0000x reference)
import functools

import jax
import jax.numpy as jnp
from jax import lax
from jax.experimental import pallas as pl
from jax.experimental.pallas import tpu as pltpu
from jax.experimental.pallas import tpu_sc as plsc

N = 10000
D = 128
R = 3
E = 100000

NPAD = 10240
NW = 32
EPW = E // NW
EPW_PAD = 3200
NCHUNK = 25
CHUNK = 128
ROWS_PER_SUB = NPAD // 16
NDRAIN = ROWS_PER_SUB // CHUNK
SHIFT = 14
MASK = (1 << SHIFT) - 1


EPWA = 3136
SBUF = EPWA + 16


def _sc_aggregate(xa, e0, e1, e2):
    mesh = plsc.VectorSubcoreMesh(core_axis_name="c", subcore_axis_name="s")

    @functools.partial(
        pl.kernel,
        out_type=(
            jax.ShapeDtypeStruct((2, R, NPAD, D), jnp.float32),
            jax.ShapeDtypeStruct((2 * R * NPAD,), jnp.float32),
        ),
        mesh=mesh,
        scratch_types=[
            pltpu.VMEM((SBUF,), jnp.int32),
            pltpu.VMEM((SBUF,), jnp.int32),
            pltpu.VMEM((NCHUNK, CHUNK), jnp.int32),
            pltpu.VMEM((NCHUNK, CHUNK), jnp.int32),
            pltpu.VMEM((CHUNK, D), jnp.float32),
            pltpu.VMEM((CHUNK, D), jnp.float32),
            pltpu.VMEM((CHUNK,), jnp.float32),
            pltpu.VMEM((ROWS_PER_SUB,), jnp.float32),
            pltpu.VMEM_SHARED((NPAD, D), jnp.float32),
            pltpu.VMEM_SHARED((NPAD,), jnp.float32),
            pltpu.SemaphoreType.DMA,
            pltpu.SemaphoreType.DMA,
            pltpu.SemaphoreType.DMA,
            pltpu.SemaphoreType.DMA,
            pltpu.SemaphoreType.DMA,
        ],
    )
    def body(xa_ref, e0_ref, e1_ref, e2_ref, msg_ref, deg_ref, sbufs, sbufd,
             srcv, dstv, rowbuf, rowbuf1, onesv, degbuf, acc, dacc, sem0,
             sem1, ssem0, ssem1, semd):
        c = lax.axis_index("c")
        s = lax.axis_index("s")
        w = c * 16 + s
        row0 = s * ROWS_PER_SUB
        base = w * EPWA
        abase = jnp.minimum(base, E - EPWA)
        off = base - abase
        cnt = jnp.minimum(EPWA, E - base)

        def _ones(k, carry):
            onesv[pl.ds(k * 16, 16)] = jnp.ones((16,), jnp.float32)
            return carry
        lax.fori_loop(0, CHUNK // 16, _ones, 0)

        for r in range(R):
            def _zb(k, carry):
                i = k // (D // 16)
                col = (k % (D // 16)) * 16
                rowbuf[i, pl.ds(col, 16)] = jnp.zeros((16,), jnp.float32)
                return carry
            lax.fori_loop(0, CHUNK * (D // 16), _zb, 0)
            for k in range(NDRAIN):
                pltpu.sync_copy(rowbuf, acc.at[pl.ds(row0 + k * CHUNK, CHUNK)])

            def _zd(k, carry):
                degbuf[pl.ds(k * 16, 16)] = jnp.zeros((16,), jnp.float32)
                return carry
            lax.fori_loop(0, ROWS_PER_SUB // 16, _zd, 0)
            pltpu.sync_copy(degbuf, dacc.at[pl.ds(row0, ROWS_PER_SUB)])
            plsc.subcore_barrier()

            eref = (e0_ref, e1_ref, e2_ref)[r]
            pltpu.sync_copy(eref.at[pl.ds(abase, EPWA)],
                            sbufs.at[pl.ds(0, EPWA)])
            pltpu.sync_copy(eref.at[pl.ds(E + abase, EPWA)],
                            sbufd.at[pl.ds(0, EPWA)])
            lanes = lax.iota(jnp.int32, 16)

            def _unpack(k, carry):
                j = k // (CHUNK // 16)
                col = (k % (CHUNK // 16)) * 16
                g = k * 16
                rd = jnp.minimum(off + g, EPWA)
                mask = (lanes + g) < cnt
                vs = sbufs[pl.ds(rd, 16)]
                vd = sbufd[pl.ds(rd, 16)]
                srcv[j, pl.ds(col, 16)] = jnp.where(mask, vs, w)
                dstv[j, pl.ds(col, 16)] = jnp.where(mask, vd, N + w)
                return carry
            lax.fori_loop(0, NCHUNK * (CHUNK // 16), _unpack, 0)

            pltpu.async_copy(xa_ref.at[srcv.at[0]], rowbuf, sem0)
            pltpu.async_copy(xa_ref.at[srcv.at[1]], rowbuf1, sem1)

            def _chunk2(i, carry):
                b = i * 2
                pltpu.make_async_copy(
                    xa_ref.at[srcv.at[b]], rowbuf, sem0).wait()
                pltpu.async_copy(rowbuf, acc.at[dstv.at[b]], ssem0, add=True)
                pltpu.async_copy(onesv, dacc.at[dstv.at[b]], semd, add=True)
                pltpu.make_async_copy(
                    xa_ref.at[srcv.at[b + 1]], rowbuf1, sem1).wait()
                pltpu.make_async_copy(
                    rowbuf, acc.at[dstv.at[b]], ssem0).wait()
                pltpu.async_copy(xa_ref.at[srcv.at[b + 2]], rowbuf, sem0)
                pltpu.async_copy(rowbuf1, acc.at[dstv.at[b + 1]], ssem1,
                                 add=True)
                pltpu.async_copy(onesv, dacc.at[dstv.at[b + 1]], semd, add=True)
                pltpu.make_async_copy(
                    rowbuf1, acc.at[dstv.at[b + 1]], ssem1).wait()

                @pl.when(b + 3 < NCHUNK)
                def _():
                    pltpu.async_copy(xa_ref.at[srcv.at[b + 3]], rowbuf1, sem1)
                return carry
            lax.fori_loop(0, (NCHUNK - 1) // 2, _chunk2, 0)

            last = NCHUNK - 1
            pltpu.make_async_copy(
                xa_ref.at[srcv.at[last]], rowbuf, sem0).wait()
            pltpu.sync_copy(rowbuf, acc.at[dstv.at[last]], add=True)
            pltpu.async_copy(onesv, dacc.at[dstv.at[last]], semd, add=True)

            def _dwait(j, carry):
                pltpu.make_async_copy(
                    onesv, dacc.at[dstv.at[j]], semd).wait()
                return carry
            lax.fori_loop(0, NCHUNK, _dwait, 0)
            plsc.subcore_barrier()

            pltpu.async_copy(acc.at[pl.ds(row0, CHUNK)], rowbuf, sem0)
            pltpu.async_copy(dacc.at[pl.ds(row0, ROWS_PER_SUB)], degbuf, semd)
            for k in range(NDRAIN):
                buf = rowbuf if k % 2 == 0 else rowbuf1
                nbuf = rowbuf1 if k % 2 == 0 else rowbuf
                sem = sem0 if k % 2 == 0 else sem1
                nsem = sem1 if k % 2 == 0 else sem0
                pltpu.make_async_copy(
                    acc.at[pl.ds(row0 + k * CHUNK, CHUNK)], buf, sem).wait()
                if k + 1 < NDRAIN:
                    pltpu.async_copy(
                        acc.at[pl.ds(row0 + (k + 1) * CHUNK, CHUNK)], nbuf,
                        nsem)
                pltpu.sync_copy(
                    buf, msg_ref.at[c, r, pl.ds(row0 + k * CHUNK, CHUNK)])
            pltpu.make_async_copy(
                dacc.at[pl.ds(row0, ROWS_PER_SUB)], degbuf, semd).wait()
            dbase = ((c * R + r) * 16 + s) * ROWS_PER_SUB
            pltpu.sync_copy(degbuf, deg_ref.at[pl.ds(dbase, ROWS_PER_SUB)])

    return body(xa, e0, e1, e2)


BN = 1024


def _tc_body(p_ref, d_ref, w_ref, b_ref, o_ref):
    h = jnp.broadcast_to(b_ref[0][None, :], (BN, D)).astype(jnp.float32)
    for r in range(R):
        msg = p_ref[0, r] + p_ref[1, r]
        deg = d_ref[r] + d_ref[R + r]
        deg = jnp.maximum(deg, 1.0).reshape(BN, 1)
        agg = msg / deg
        h = h + jnp.dot(agg, w_ref[r], preferred_element_type=jnp.float32)
    o_ref[...] = h


def _tc_combine(msg, deg, W, h_bias):
    return pl.pallas_call(
        _tc_body,
        grid=(NPAD // BN,),
        in_specs=[
            pl.BlockSpec((2, R, BN, D), lambda i: (0, 0, i, 0)),
            pl.BlockSpec((2 * R, BN), lambda i: (0, i)),
            pl.BlockSpec((R, D, D), lambda i: (0, 0, 0)),
            pl.BlockSpec((1, D), lambda i: (0, 0)),
        ],
        out_specs=pl.BlockSpec((BN, D), lambda i: (i, 0)),
        out_shape=jax.ShapeDtypeStruct((N, D), jnp.float32),
    )(msg, deg, W, h_bias.reshape(1, D))


def kernel(x, edge_index_rel0, edge_index_rel1, edge_index_rel2, W, h_bias):
    msg, deg = _sc_aggregate(x, edge_index_rel0.reshape(2 * E),
                             edge_index_rel1.reshape(2 * E),
                             edge_index_rel2.reshape(2 * E))
    return _tc_combine(msg, deg.reshape(2 * R, NPAD), W, h_bias)


# speedup vs baseline: 1.1439x; 1.1439x over previous
import functools

import jax
import jax.numpy as jnp
from jax import lax
from jax.experimental import pallas as pl
from jax.experimental.pallas import tpu as pltpu
from jax.experimental.pallas import tpu_sc as plsc

N = 10000
D = 128
R = 3
E = 100000

NPAD = 10240
NW = 32
EPW = E // NW
EPW_PAD = 3200
NCHUNK = 25
CHUNK = 128
ROWS_PER_SUB = NPAD // 16
NDRAIN = ROWS_PER_SUB // CHUNK
SHIFT = 14
MASK = (1 << SHIFT) - 1


def _sc_aggregate(xa, edges):
    mesh = plsc.VectorSubcoreMesh(core_axis_name="c", subcore_axis_name="s")

    @functools.partial(
        pl.kernel,
        out_type=(
            jax.ShapeDtypeStruct((2, R, NPAD, D), jnp.float32),
            jax.ShapeDtypeStruct((2 * R * NPAD,), jnp.float32),
        ),
        mesh=mesh,
        scratch_types=[
            pltpu.VMEM((NCHUNK, CHUNK), jnp.int32),
            pltpu.VMEM((NCHUNK, CHUNK), jnp.int32),
            pltpu.VMEM((NCHUNK, CHUNK), jnp.int32),
            pltpu.VMEM((CHUNK, D), jnp.float32),
            pltpu.VMEM((CHUNK, D), jnp.float32),
            pltpu.VMEM((CHUNK,), jnp.float32),
            pltpu.VMEM((ROWS_PER_SUB,), jnp.float32),
            pltpu.VMEM_SHARED((NPAD, D), jnp.float32),
            pltpu.VMEM_SHARED((NPAD,), jnp.float32),
            pltpu.SemaphoreType.DMA,
            pltpu.SemaphoreType.DMA,
            pltpu.SemaphoreType.DMA,
            pltpu.SemaphoreType.DMA,
            pltpu.SemaphoreType.DMA,
        ],
    )
    def body(xa_ref, edges_ref, msg_ref, deg_ref, packedv, srcv, dstv,
             rowbuf, rowbuf1, onesv, degbuf, acc, dacc, sem0, sem1, ssem0,
             ssem1, semd):
        c = lax.axis_index("c")
        s = lax.axis_index("s")
        w = c * 16 + s
        row0 = s * ROWS_PER_SUB

        def _ones(k, carry):
            onesv[pl.ds(k * 16, 16)] = jnp.ones((16,), jnp.float32)
            return carry
        lax.fori_loop(0, CHUNK // 16, _ones, 0)

        for r in range(R):
            def _zb(k, carry):
                i = k // (D // 16)
                col = (k % (D // 16)) * 16
                rowbuf[i, pl.ds(col, 16)] = jnp.zeros((16,), jnp.float32)
                return carry
            lax.fori_loop(0, CHUNK * (D // 16), _zb, 0)
            for k in range(NDRAIN):
                pltpu.sync_copy(rowbuf, acc.at[pl.ds(row0 + k * CHUNK, CHUNK)])

            def _zd(k, carry):
                degbuf[pl.ds(k * 16, 16)] = jnp.zeros((16,), jnp.float32)
                return carry
            lax.fori_loop(0, ROWS_PER_SUB // 16, _zd, 0)
            pltpu.sync_copy(degbuf, dacc.at[pl.ds(row0, ROWS_PER_SUB)])
            plsc.subcore_barrier()

            pltpu.sync_copy(edges_ref.at[r, w], packedv)

            def _unpack(k, carry):
                j = k // (CHUNK // 16)
                col = (k % (CHUNK // 16)) * 16
                v = packedv[j, pl.ds(col, 16)]
                srcv[j, pl.ds(col, 16)] = lax.shift_right_logical(v, SHIFT)
                dstv[j, pl.ds(col, 16)] = lax.bitwise_and(v, MASK)
                return carry
            lax.fori_loop(0, NCHUNK * (CHUNK // 16), _unpack, 0)

            pltpu.async_copy(xa_ref.at[srcv.at[0]], rowbuf, sem0)
            pltpu.async_copy(xa_ref.at[srcv.at[1]], rowbuf1, sem1)

            def _chunk2(i, carry):
                b = i * 2
                pltpu.make_async_copy(
                    xa_ref.at[srcv.at[b]], rowbuf, sem0).wait()
                pltpu.async_copy(rowbuf, acc.at[dstv.at[b]], ssem0, add=True)
                pltpu.async_copy(onesv, dacc.at[dstv.at[b]], semd, add=True)
                pltpu.make_async_copy(
                    xa_ref.at[srcv.at[b + 1]], rowbuf1, sem1).wait()
                pltpu.make_async_copy(
                    rowbuf, acc.at[dstv.at[b]], ssem0).wait()
                pltpu.async_copy(xa_ref.at[srcv.at[b + 2]], rowbuf, sem0)
                pltpu.async_copy(rowbuf1, acc.at[dstv.at[b + 1]], ssem1,
                                 add=True)
                pltpu.async_copy(onesv, dacc.at[dstv.at[b + 1]], semd, add=True)
                pltpu.make_async_copy(
                    rowbuf1, acc.at[dstv.at[b + 1]], ssem1).wait()

                @pl.when(b + 3 < NCHUNK)
                def _():
                    pltpu.async_copy(xa_ref.at[srcv.at[b + 3]], rowbuf1, sem1)
                return carry
            lax.fori_loop(0, (NCHUNK - 1) // 2, _chunk2, 0)

            last = NCHUNK - 1
            pltpu.make_async_copy(
                xa_ref.at[srcv.at[last]], rowbuf, sem0).wait()
            pltpu.sync_copy(rowbuf, acc.at[dstv.at[last]], add=True)
            pltpu.async_copy(onesv, dacc.at[dstv.at[last]], semd, add=True)

            def _dwait(j, carry):
                pltpu.make_async_copy(
                    onesv, dacc.at[dstv.at[j]], semd).wait()
                return carry
            lax.fori_loop(0, NCHUNK, _dwait, 0)
            plsc.subcore_barrier()

            pltpu.async_copy(acc.at[pl.ds(row0, CHUNK)], rowbuf, sem0)
            pltpu.async_copy(dacc.at[pl.ds(row0, ROWS_PER_SUB)], degbuf, semd)
            for k in range(NDRAIN):
                buf = rowbuf if k % 2 == 0 else rowbuf1
                nbuf = rowbuf1 if k % 2 == 0 else rowbuf
                sem = sem0 if k % 2 == 0 else sem1
                nsem = sem1 if k % 2 == 0 else sem0
                pltpu.make_async_copy(
                    acc.at[pl.ds(row0 + k * CHUNK, CHUNK)], buf, sem).wait()
                if k + 1 < NDRAIN:
                    pltpu.async_copy(
                        acc.at[pl.ds(row0 + (k + 1) * CHUNK, CHUNK)], nbuf,
                        nsem)
                pltpu.sync_copy(
                    buf, msg_ref.at[c, r, pl.ds(row0 + k * CHUNK, CHUNK)])
            pltpu.make_async_copy(
                dacc.at[pl.ds(row0, ROWS_PER_SUB)], degbuf, semd).wait()
            dbase = ((c * R + r) * 16 + s) * ROWS_PER_SUB
            pltpu.sync_copy(degbuf, deg_ref.at[pl.ds(dbase, ROWS_PER_SUB)])

    return body(xa, edges)


BN = 1024


def _tc_body(p_ref, d_ref, w_ref, b_ref, o_ref):
    h = jnp.broadcast_to(b_ref[0][None, :], (BN, D)).astype(jnp.float32)
    for r in range(R):
        msg = p_ref[0, r] + p_ref[1, r]
        deg = d_ref[r] + d_ref[R + r]
        deg = jnp.maximum(deg, 1.0).reshape(BN, 1)
        agg = msg / deg
        h = h + jnp.dot(agg, w_ref[r], preferred_element_type=jnp.float32)
    o_ref[...] = h


def _tc_combine(msg, deg, W, h_bias):
    return pl.pallas_call(
        _tc_body,
        grid=(NPAD // BN,),
        in_specs=[
            pl.BlockSpec((2, R, BN, D), lambda i: (0, 0, i, 0)),
            pl.BlockSpec((2 * R, BN), lambda i: (0, i)),
            pl.BlockSpec((R, D, D), lambda i: (0, 0, 0)),
            pl.BlockSpec((1, D), lambda i: (0, 0)),
        ],
        out_specs=pl.BlockSpec((BN, D), lambda i: (i, 0)),
        out_shape=jax.ShapeDtypeStruct((N, D), jnp.float32),
    )(msg, deg, W, h_bias.reshape(1, D))


def kernel(x, edge_index_rel0, edge_index_rel1, edge_index_rel2, W, h_bias):
    wids = jnp.arange(NW, dtype=jnp.int32).reshape(1, NW, 1)
    pad = jnp.broadcast_to((wids << SHIFT) | (N + wids),
                           (R, NW, EPW_PAD - EPW))
    srcs = jnp.stack([edge_index_rel0[0], edge_index_rel1[0],
                      edge_index_rel2[0]]).reshape(R, NW, EPW)
    dsts = jnp.stack([edge_index_rel0[1], edge_index_rel1[1],
                      edge_index_rel2[1]]).reshape(R, NW, EPW)
    edges = jnp.concatenate([(srcs << SHIFT) | dsts, pad], axis=2)
    edges = edges.reshape(R, NW, NCHUNK, CHUNK)

    msg, deg = _sc_aggregate(x, edges)
    return _tc_combine(msg, deg.reshape(2 * R, NPAD), W, h_bias)


# speedup vs baseline: 1.1590x; 1.0132x over previous
import functools

import jax
import jax.numpy as jnp
from jax import lax
from jax.experimental import pallas as pl
from jax.experimental.pallas import tpu as pltpu
from jax.experimental.pallas import tpu_sc as plsc

N = 10000
D = 128
R = 3
E = 100000

NPAD = 10240
NW = 32
EPW = E // NW
EPW_PAD = 3200
NCHUNK = 25
CHUNK = 128
ROWS_PER_SUB = NPAD // 16
NDRAIN = ROWS_PER_SUB // CHUNK
SHIFT = 14
MASK = (1 << SHIFT) - 1


def _sc_aggregate(xa, edges):
    mesh = plsc.VectorSubcoreMesh(core_axis_name="c", subcore_axis_name="s")

    @functools.partial(
        pl.kernel,
        out_type=(
            jax.ShapeDtypeStruct((2, R, NPAD, D), jnp.float32),
            jax.ShapeDtypeStruct((2 * R * NPAD,), jnp.float32),
        ),
        mesh=mesh,
        scratch_types=[
            pltpu.VMEM((NCHUNK, CHUNK), jnp.int32),
            pltpu.VMEM((NCHUNK, CHUNK), jnp.int32),
            pltpu.VMEM((NCHUNK, CHUNK), jnp.int32),
            pltpu.VMEM((CHUNK, D), jnp.float32),
            pltpu.VMEM((CHUNK, D), jnp.float32),
            pltpu.VMEM((CHUNK,), jnp.float32),
            pltpu.VMEM((ROWS_PER_SUB,), jnp.float32),
            pltpu.VMEM_SHARED((NPAD, D), jnp.float32),
            pltpu.VMEM_SHARED((NPAD,), jnp.float32),
            pltpu.SemaphoreType.DMA,
            pltpu.SemaphoreType.DMA,
            pltpu.SemaphoreType.DMA,
            pltpu.SemaphoreType.DMA,
            pltpu.SemaphoreType.DMA,
        ],
    )
    def body(xa_ref, edges_ref, msg_ref, deg_ref, packedv, srcv, dstv,
             rowbuf, rowbuf1, onesv, degbuf, acc, dacc, sem0, sem1, ssem0,
             ssem1, semd):
        c = lax.axis_index("c")
        s = lax.axis_index("s")
        w = c * 16 + s
        row0 = s * ROWS_PER_SUB

        def _ones(k, carry):
            onesv[pl.ds(k * 16, 16)] = jnp.ones((16,), jnp.float32)
            return carry
        lax.fori_loop(0, CHUNK // 16, _ones, 0)

        for r in range(R):
            def _zb(k, carry):
                i = k // (D // 16)
                col = (k % (D // 16)) * 16
                rowbuf[i, pl.ds(col, 16)] = jnp.zeros((16,), jnp.float32)
                return carry
            lax.fori_loop(0, CHUNK * (D // 16), _zb, 0)
            for k in range(NDRAIN):
                pltpu.sync_copy(rowbuf, acc.at[pl.ds(row0 + k * CHUNK, CHUNK)])

            def _zd(k, carry):
                degbuf[pl.ds(k * 16, 16)] = jnp.zeros((16,), jnp.float32)
                return carry
            lax.fori_loop(0, ROWS_PER_SUB // 16, _zd, 0)
            pltpu.sync_copy(degbuf, dacc.at[pl.ds(row0, ROWS_PER_SUB)])
            plsc.subcore_barrier()

            pltpu.sync_copy(edges_ref.at[r, w], packedv)

            def _unpack(k, carry):
                j = k // (CHUNK // 16)
                col = (k % (CHUNK // 16)) * 16
                v = packedv[j, pl.ds(col, 16)]
                srcv[j, pl.ds(col, 16)] = lax.shift_right_logical(v, SHIFT)
                dstv[j, pl.ds(col, 16)] = lax.bitwise_and(v, MASK)
                return carry
            lax.fori_loop(0, NCHUNK * (CHUNK // 16), _unpack, 0)

            pltpu.async_copy(xa_ref.at[srcv.at[0]], rowbuf, sem0)
            pltpu.async_copy(xa_ref.at[srcv.at[1]], rowbuf1, sem1)

            def _chunk2(i, carry):
                b = i * 2
                pltpu.make_async_copy(
                    xa_ref.at[srcv.at[b]], rowbuf, sem0).wait()
                pltpu.async_copy(rowbuf, acc.at[dstv.at[b]], ssem0, add=True)
                pltpu.async_copy(onesv, dacc.at[dstv.at[b]], semd, add=True)
                pltpu.make_async_copy(
                    xa_ref.at[srcv.at[b + 1]], rowbuf1, sem1).wait()
                pltpu.make_async_copy(
                    rowbuf, acc.at[dstv.at[b]], ssem0).wait()
                pltpu.async_copy(xa_ref.at[srcv.at[b + 2]], rowbuf, sem0)
                pltpu.async_copy(rowbuf1, acc.at[dstv.at[b + 1]], ssem1,
                                 add=True)
                pltpu.async_copy(onesv, dacc.at[dstv.at[b + 1]], semd, add=True)
                pltpu.make_async_copy(
                    rowbuf1, acc.at[dstv.at[b + 1]], ssem1).wait()

                @pl.when(b + 3 < NCHUNK)
                def _():
                    pltpu.async_copy(xa_ref.at[srcv.at[b + 3]], rowbuf1, sem1)
                return carry
            lax.fori_loop(0, (NCHUNK - 1) // 2, _chunk2, 0)

            last = NCHUNK - 1
            pltpu.make_async_copy(
                xa_ref.at[srcv.at[last]], rowbuf, sem0).wait()
            pltpu.sync_copy(rowbuf, acc.at[dstv.at[last]], add=True)
            pltpu.async_copy(onesv, dacc.at[dstv.at[last]], semd, add=True)

            def _dwait(j, carry):
                pltpu.make_async_copy(
                    onesv, dacc.at[dstv.at[j]], semd).wait()
                return carry
            lax.fori_loop(0, NCHUNK, _dwait, 0)
            plsc.subcore_barrier()

            pltpu.async_copy(acc.at[pl.ds(row0, CHUNK)], rowbuf, sem0)
            pltpu.async_copy(dacc.at[pl.ds(row0, ROWS_PER_SUB)], degbuf, semd)
            for k in range(NDRAIN):
                buf = rowbuf if k % 2 == 0 else rowbuf1
                nbuf = rowbuf1 if k % 2 == 0 else rowbuf
                sem = sem0 if k % 2 == 0 else sem1
                nsem = sem1 if k % 2 == 0 else sem0
                pltpu.make_async_copy(
                    acc.at[pl.ds(row0 + k * CHUNK, CHUNK)], buf, sem).wait()
                if k + 1 < NDRAIN:
                    pltpu.async_copy(
                        acc.at[pl.ds(row0 + (k + 1) * CHUNK, CHUNK)], nbuf,
                        nsem)
                pltpu.sync_copy(
                    buf, msg_ref.at[c, r, pl.ds(row0 + k * CHUNK, CHUNK)])
            pltpu.make_async_copy(
                dacc.at[pl.ds(row0, ROWS_PER_SUB)], degbuf, semd).wait()
            dbase = ((c * R + r) * 16 + s) * ROWS_PER_SUB
            pltpu.sync_copy(degbuf, deg_ref.at[pl.ds(dbase, ROWS_PER_SUB)])

    return body(xa, edges)


BN = 2048


def _tc_body(p_ref, d_ref, w_ref, b_ref, o_ref):
    h = jnp.broadcast_to(b_ref[0][None, :], (BN, D)).astype(jnp.float32)
    for r in range(R):
        msg = p_ref[0, r] + p_ref[1, r]
        deg = d_ref[r] + d_ref[R + r]
        deg = jnp.maximum(deg, 1.0).reshape(BN, 1)
        agg = msg / deg
        h = h + jnp.dot(agg, w_ref[r], preferred_element_type=jnp.float32)
    o_ref[...] = h


def _tc_combine(msg, deg, W, h_bias):
    return pl.pallas_call(
        _tc_body,
        grid=(NPAD // BN,),
        in_specs=[
            pl.BlockSpec((2, R, BN, D), lambda i: (0, 0, i, 0)),
            pl.BlockSpec((2 * R, BN), lambda i: (0, i)),
            pl.BlockSpec((R, D, D), lambda i: (0, 0, 0)),
            pl.BlockSpec((1, D), lambda i: (0, 0)),
        ],
        out_specs=pl.BlockSpec((BN, D), lambda i: (i, 0)),
        out_shape=jax.ShapeDtypeStruct((N, D), jnp.float32),
    )(msg, deg, W, h_bias.reshape(1, D))


def kernel(x, edge_index_rel0, edge_index_rel1, edge_index_rel2, W, h_bias):
    wids = jnp.arange(NW, dtype=jnp.int32).reshape(1, NW, 1)
    pad = jnp.broadcast_to((wids << SHIFT) | (N + wids),
                           (R, NW, EPW_PAD - EPW))
    srcs = jnp.stack([edge_index_rel0[0], edge_index_rel1[0],
                      edge_index_rel2[0]]).reshape(R, NW, EPW)
    dsts = jnp.stack([edge_index_rel0[1], edge_index_rel1[1],
                      edge_index_rel2[1]]).reshape(R, NW, EPW)
    edges = jnp.concatenate([(srcs << SHIFT) | dsts, pad], axis=2)
    edges = edges.reshape(R, NW, NCHUNK, CHUNK)

    msg, deg = _sc_aggregate(x, edges)
    return _tc_combine(msg, deg.reshape(2 * R, NPAD), W, h_bias)
